# baseline (speedup 1.0000x reference)
import functools

import jax
import jax.numpy as jnp
import numpy as np
from jax import lax
from jax.experimental import pallas as pl
from jax.experimental.pallas import tpu as pltpu
from jax.experimental.pallas import tpu_sc as plsc

NNODE = 10000
NEDGE = 320000
NFEAT = 128
NGRAPH = 64
RATIO = 0.5

NC, NS, LANES = 2, 16, 16
NW = NC * NS
EC = 128
NCH = -(-NEDGE // (NW * EC))
ZC = 80

_RPAD = {10000: 10240, 5000: 5120, 2500: 2560, 1250: 1280}


def _rpad(k):
    return _RPAD[k]


@functools.lru_cache(maxsize=None)
def _propagate(R, F):
    RT = R // NS
    mesh = plsc.VectorSubcoreMesh(core_axis_name="c", subcore_axis_name="s")

    @functools.partial(
        pl.kernel,
        out_type=jax.ShapeDtypeStruct((NC, R, F), jnp.float32),
        mesh=mesh,
        scratch_types=[
            pltpu.VMEM((EC,), jnp.int32),
            pltpu.VMEM((EC,), jnp.int32),
            pltpu.VMEM((EC, F), jnp.float32),
            pltpu.VMEM((ZC, F), jnp.float32),
            pltpu.VMEM_SHARED((R, F), jnp.float32),
            pltpu.SemaphoreType.DMA,
        ],
    )
    def kern(g_h, src_h, dst_h, out_h, sidx, didx, rows, zbuf, acc_sh, sem):
        cid = lax.axis_index("c")
        sid = lax.axis_index("s")
        tid = sid * NC + cid
        zv = jnp.zeros((LANES,), jnp.float32)

        def zrow(i, _):
            for jf in range(F // LANES):
                zbuf[i, pl.ds(jf * LANES, LANES)] = zv
            return 0

        lax.fori_loop(0, ZC, zrow, 0)

        def zslab(i, _):
            pltpu.sync_copy(zbuf, acc_sh.at[pl.ds(sid * RT + i * ZC, ZC)])
            return 0

        lax.fori_loop(0, RT // ZC, zslab, 0)
        plsc.subcore_barrier()

        def echunk(j, _):
            pltpu.sync_copy(src_h.at[tid, j], sidx)
            pltpu.sync_copy(dst_h.at[tid, j], didx)
            pltpu.async_copy(g_h.at[sidx], rows, sem).wait()
            pltpu.sync_copy(rows, acc_sh.at[didx], add=True)
            return 0

        lax.fori_loop(0, NCH, echunk, 0)
        plsc.subcore_barrier()

        def wb(i, _):
            sl = pl.ds(sid * RT + i * ZC, ZC)
            pltpu.sync_copy(acc_sh.at[sl], out_h.at[cid, sl])
            return 0

        lax.fori_loop(0, RT // ZC, wb, 0)

    return kern


BR = 256


def _mm_body(x_ref, w_ref, b_ref, ri_ref, ro_ref, o_ref):
    xb = x_ref[...] * ri_ref[...]
    acc = jnp.dot(xb, w_ref[...], preferred_element_type=jnp.float32)
    o_ref[...] = (acc + b_ref[0:1, :]) * ro_ref[...]


def _mm_scale(x, W, b, rs_in, rs_out):
    R, K = x.shape
    N = W.shape[1]
    b8 = jnp.broadcast_to(b[None, :], (8, N))
    return pl.pallas_call(
        _mm_body,
        grid=(R // BR,),
        in_specs=[
            pl.BlockSpec((BR, K), lambda i: (i, 0)),
            pl.BlockSpec((K, N), lambda i: (0, 0)),
            pl.BlockSpec((8, N), lambda i: (0, 0)),
            pl.BlockSpec((BR, 1), lambda i: (i, 0)),
            pl.BlockSpec((BR, 1), lambda i: (i, 0)),
        ],
        out_specs=pl.BlockSpec((BR, N), lambda i: (i, 0)),
        out_shape=jax.ShapeDtypeStruct((R, N), jnp.float32),
    )(x, W, b8, rs_in, rs_out)


def _combine_body(a_ref, g_ref, d_ref, o_ref):
    o_ref[...] = jnp.maximum((a_ref[0] + a_ref[1] + g_ref[...]) * d_ref[...], 0.0)


def _combine(acc, g, dinv):
    R, F = g.shape
    return pl.pallas_call(
        _combine_body,
        grid=(R // BR,),
        in_specs=[
            pl.BlockSpec((NC, BR, F), lambda i: (0, i, 0)),
            pl.BlockSpec((BR, F), lambda i: (i, 0)),
            pl.BlockSpec((BR, 1), lambda i: (i, 0)),
        ],
        out_specs=pl.BlockSpec((BR, F), lambda i: (i, 0)),
        out_shape=jax.ShapeDtypeStruct((R, F), jnp.float32),
    )(acc, g, dinv)


def _pad_edges(src, dst, trash):
    tot = NW * NCH * EC
    sp = jnp.full((tot,), trash, jnp.int32).at[:NEDGE].set(src)
    dp = jnp.full((tot,), trash, jnp.int32).at[:NEDGE].set(dst)
    return sp.reshape(NW, NCH, EC), dp.reshape(NW, NCH, EC)


def _degree(k, R, src3, dst3):
    rows = lax.broadcasted_iota(jnp.int32, (R, 1), 0)
    gdeg = jnp.broadcast_to((rows < k).astype(jnp.float32), (R, NFEAT))
    acc = _propagate(R, NFEAT)(gdeg, src3, dst3)
    deg = acc[0, :, 0] + acc[1, :, 0] + 1.0
    dinv = jnp.where(rows[:, 0] < k, lax.rsqrt(deg), 0.0)
    return dinv[:, None]


def _gcn(x_pad, W, b, dinv, src3, dst3, rs_in=None):
    R = x_pad.shape[0]
    if rs_in is None:
        rs_in = jnp.ones((R, 1), jnp.float32)
    g = _mm_scale(x_pad, W, b, rs_in, dinv)
    acc = _propagate(R, NFEAT)(g, src3, dst3)
    return _combine(acc, g, dinv)


def _gap(x, batch):
    s = jax.ops.segment_sum(x, batch, num_segments=NGRAPH)
    c = jax.ops.segment_sum(jnp.ones((x.shape[0],), jnp.float32), batch,
                            num_segments=NGRAPH)
    return s / jnp.maximum(c, 1.0)[:, None]


def _gmp(x, batch):
    m = jax.ops.segment_max(x, batch, num_segments=NGRAPH)
    c = jax.ops.segment_sum(jnp.ones((x.shape[0],), jnp.float32), batch,
                            num_segments=NGRAPH)
    return jnp.where((c > 0.0)[:, None], m, 0.0)


def _mvpool_jax(y_pad, kprev, src, dst, bt, p, k):
    x = y_pad[:kprev]
    score = jnp.tanh((x @ p) / (jnp.linalg.norm(p) + 1e-12))
    vals, perm = lax.top_k(score, k)
    x_new = x[perm] * vals[:, None]
    newid = jnp.full((kprev + 1,), k, jnp.int32).at[perm].set(
        jnp.arange(k, dtype=jnp.int32))
    return x_new, newid[src], newid[dst], bt[perm]


def kernel(x, edge_index, batch, W1, b1, W2, b2, W3, b3, W7, b7, W8, b8,
           p1, p2, p3):
    src = edge_index[0]
    dst = edge_index[1]
    R0 = _rpad(NNODE)
    x_pad = jnp.zeros((R0, NFEAT), jnp.float32).at[:NNODE].set(x)
    src3, dst3 = _pad_edges(src, dst, NNODE)
    dinv0 = _degree(NNODE, R0, src3, dst3)

    dkey = jax.random.key(7)
    augs = []
    for i in range(10):
        keep = (jax.random.uniform(jax.random.fold_in(dkey, i),
                                   (NNODE,)) > 0.1).astype(jnp.float32)
        keep_pad = jnp.zeros((R0, 1), jnp.float32).at[:NNODE, 0].set(keep)
        y = _gcn(x_pad, W7, b7, dinv0, src3, dst3, rs_in=keep_pad)
        y = _gcn(y, W8, b8, dinv0, src3, dst3)
        yv = y[:NNODE]
        augs.append(_gmp(yv, batch))
        augs.append(_gap(yv, batch))
    augs = jnp.concatenate(augs, axis=1)

    h = _gcn(x_pad, W1, b1, dinv0, src3, dst3)
    ks = [NNODE]
    for r in range(3):
        ks.append(int(np.ceil(RATIO * ks[-1])))
    bt = batch
    xs = []
    for lvl, (p, W, b) in enumerate(
            [(p1, W2, b2), (p2, W3, b3), (p3, None, None)]):
        kprev, k = ks[lvl], ks[lvl + 1]
        x_new, src, dst, bt = _mvpool_jax(h, kprev, src, dst, bt, p, k)
        xs.append(jnp.concatenate([_gmp(x_new, bt), _gap(x_new, bt)], axis=1))
        if W is None:
            break
        R = _rpad(k)
        h_pad = jnp.zeros((R, NFEAT), jnp.float32).at[:k].set(x_new)
        src3, dst3 = _pad_edges(src, dst, k)
        dinv = _degree(k, R, src3, dst3)
        h = _gcn(h_pad, W, b, dinv, src3, dst3)

    out = jax.nn.relu(xs[0]) + jax.nn.relu(xs[1]) + jax.nn.relu(xs[2])
    return (out, augs)


# speedup vs baseline: 3.5847x; 3.5847x over previous
import functools

import jax
import jax.numpy as jnp
import numpy as np
from jax import lax
from jax.experimental import pallas as pl
from jax.experimental.pallas import tpu as pltpu
from jax.experimental.pallas import tpu_sc as plsc

NNODE = 10000
NEDGE = 320000
NFEAT = 128
NGRAPH = 64
RATIO = 0.5

NC, NS, LANES = 2, 16, 16
NW = NC * NS
EC = 128
NCH = -(-NEDGE // (NW * EC))
ZC = 80

_RPAD = {10000: 10240, 5000: 6400, 2500: 3840, 1250: 2560}


def _rpad(k):
    return _RPAD[k]


@functools.lru_cache(maxsize=None)
def _propagate(R, F):
    RT = R // NS
    mesh = plsc.VectorSubcoreMesh(core_axis_name="c", subcore_axis_name="s")

    @functools.partial(
        pl.kernel,
        out_type=jax.ShapeDtypeStruct((NC, R, F), jnp.float32),
        mesh=mesh,
        scratch_types=[
            pltpu.VMEM((EC,), jnp.int32),
            pltpu.VMEM((EC,), jnp.int32),
            pltpu.VMEM((EC, F), jnp.float32),
            pltpu.VMEM((ZC, F), jnp.float32),
            pltpu.VMEM_SHARED((R, F), jnp.float32),
            pltpu.SemaphoreType.DMA,
        ],
    )
    def kern(g_h, src_h, dst_h, out_h, sidx, didx, rows, zbuf, acc_sh, sem):
        cid = lax.axis_index("c")
        sid = lax.axis_index("s")
        tid = sid * NC + cid
        zv = jnp.zeros((LANES,), jnp.float32)

        def zrow(i, _):
            for jf in range(F // LANES):
                zbuf[i, pl.ds(jf * LANES, LANES)] = zv
            return 0

        lax.fori_loop(0, ZC, zrow, 0)

        def zslab(i, _):
            pltpu.sync_copy(zbuf, acc_sh.at[pl.ds(sid * RT + i * ZC, ZC)])
            return 0

        lax.fori_loop(0, RT // ZC, zslab, 0)
        plsc.subcore_barrier()

        def echunk(j, _):
            pltpu.sync_copy(src_h.at[tid, j], sidx)
            pltpu.sync_copy(dst_h.at[tid, j], didx)
            pltpu.async_copy(g_h.at[sidx], rows, sem).wait()
            pltpu.sync_copy(rows, acc_sh.at[didx], add=True)
            return 0

        lax.fori_loop(0, NCH, echunk, 0)
        plsc.subcore_barrier()

        def wb(i, _):
            sl = pl.ds(sid * RT + i * ZC, ZC)
            pltpu.sync_copy(acc_sh.at[sl], out_h.at[cid, sl])
            return 0

        lax.fori_loop(0, RT // ZC, wb, 0)

    return kern


BR = 256


def _mm_body(x_ref, w_ref, b_ref, ri_ref, ro_ref, o_ref):
    xb = x_ref[...] * ri_ref[...]
    acc = jnp.dot(xb, w_ref[...], preferred_element_type=jnp.float32)
    o_ref[...] = (acc + b_ref[0:1, :]) * ro_ref[...]


def _mm_scale(x, W, b, rs_in, rs_out):
    R, K = x.shape
    N = W.shape[1]
    b8 = jnp.broadcast_to(b[None, :], (8, N))
    return pl.pallas_call(
        _mm_body,
        grid=(R // BR,),
        in_specs=[
            pl.BlockSpec((BR, K), lambda i: (i, 0)),
            pl.BlockSpec((K, N), lambda i: (0, 0)),
            pl.BlockSpec((8, N), lambda i: (0, 0)),
            pl.BlockSpec((BR, 1), lambda i: (i, 0)),
            pl.BlockSpec((BR, 1), lambda i: (i, 0)),
        ],
        out_specs=pl.BlockSpec((BR, N), lambda i: (i, 0)),
        out_shape=jax.ShapeDtypeStruct((R, N), jnp.float32),
    )(x, W, b8, rs_in, rs_out)


def _combine_body(a_ref, g_ref, d_ref, o_ref):
    o_ref[...] = jnp.maximum((a_ref[0] + a_ref[1] + g_ref[...]) * d_ref[...], 0.0)


def _combine(acc, g, dinv):
    R, F = g.shape
    return pl.pallas_call(
        _combine_body,
        grid=(R // BR,),
        in_specs=[
            pl.BlockSpec((NC, BR, F), lambda i: (0, i, 0)),
            pl.BlockSpec((BR, F), lambda i: (i, 0)),
            pl.BlockSpec((BR, 1), lambda i: (i, 0)),
        ],
        out_specs=pl.BlockSpec((BR, F), lambda i: (i, 0)),
        out_shape=jax.ShapeDtypeStruct((R, F), jnp.float32),
    )(acc, g, dinv)


def _pad_edges(src, dst, k, R):
    tot = NW * NCH * EC
    sp = jnp.full((tot,), k, jnp.int32).at[:NEDGE].set(src)
    dp = jnp.full((tot,), k, jnp.int32).at[:NEDGE].set(dst)
    e = jnp.arange(tot, dtype=jnp.int32)
    sp = jnp.where(sp >= k, k + e % (R - k), sp)
    dp = jnp.where(dp >= k, k + (e + 7) % (R - k), dp)
    return sp.reshape(NW, NCH, EC), dp.reshape(NW, NCH, EC)


def _degree(k, R, src3, dst3):
    rows = lax.broadcasted_iota(jnp.int32, (R, 1), 0)
    gdeg = jnp.broadcast_to((rows < k).astype(jnp.float32), (R, NFEAT))
    acc = _propagate(R, NFEAT)(gdeg, src3, dst3)
    deg = acc[0, :, 0] + acc[1, :, 0] + 1.0
    dinv = jnp.where(rows[:, 0] < k, lax.rsqrt(deg), 0.0)
    return dinv[:, None]


def _gcn(x_pad, W, b, dinv, src3, dst3, rs_in=None):
    R = x_pad.shape[0]
    if rs_in is None:
        rs_in = jnp.ones((R, 1), jnp.float32)
    g = _mm_scale(x_pad, W, b, rs_in, dinv)
    acc = _propagate(R, NFEAT)(g, src3, dst3)
    return _combine(acc, g, dinv)


def _gap(x, batch):
    s = jax.ops.segment_sum(x, batch, num_segments=NGRAPH)
    c = jax.ops.segment_sum(jnp.ones((x.shape[0],), jnp.float32), batch,
                            num_segments=NGRAPH)
    return s / jnp.maximum(c, 1.0)[:, None]


def _gmp(x, batch):
    m = jax.ops.segment_max(x, batch, num_segments=NGRAPH)
    c = jax.ops.segment_sum(jnp.ones((x.shape[0],), jnp.float32), batch,
                            num_segments=NGRAPH)
    return jnp.where((c > 0.0)[:, None], m, 0.0)


def _mvpool_jax(y_pad, kprev, src, dst, bt, p, k):
    x = y_pad[:kprev]
    score = jnp.tanh((x @ p) / (jnp.linalg.norm(p) + 1e-12))
    vals, perm = lax.top_k(score, k)
    x_new = x[perm] * vals[:, None]
    newid = jnp.full((kprev + 1,), k, jnp.int32).at[perm].set(
        jnp.arange(k, dtype=jnp.int32))
    return x_new, newid[src], newid[dst], bt[perm]


def kernel(x, edge_index, batch, W1, b1, W2, b2, W3, b3, W7, b7, W8, b8,
           p1, p2, p3):
    src = edge_index[0]
    dst = edge_index[1]
    R0 = _rpad(NNODE)
    x_pad = jnp.zeros((R0, NFEAT), jnp.float32).at[:NNODE].set(x)
    src3, dst3 = _pad_edges(src, dst, NNODE, R0)
    dinv0 = _degree(NNODE, R0, src3, dst3)

    dkey = jax.random.key(7)
    augs = []
    for i in range(10):
        keep = (jax.random.uniform(jax.random.fold_in(dkey, i),
                                   (NNODE,)) > 0.1).astype(jnp.float32)
        keep_pad = jnp.zeros((R0, 1), jnp.float32).at[:NNODE, 0].set(keep)
        y = _gcn(x_pad, W7, b7, dinv0, src3, dst3, rs_in=keep_pad)
        y = _gcn(y, W8, b8, dinv0, src3, dst3)
        yv = y[:NNODE]
        augs.append(_gmp(yv, batch))
        augs.append(_gap(yv, batch))
    augs = jnp.concatenate(augs, axis=1)

    h = _gcn(x_pad, W1, b1, dinv0, src3, dst3)
    ks = [NNODE]
    for r in range(3):
        ks.append(int(np.ceil(RATIO * ks[-1])))
    bt = batch
    xs = []
    for lvl, (p, W, b) in enumerate(
            [(p1, W2, b2), (p2, W3, b3), (p3, None, None)]):
        kprev, k = ks[lvl], ks[lvl + 1]
        x_new, src, dst, bt = _mvpool_jax(h, kprev, src, dst, bt, p, k)
        xs.append(jnp.concatenate([_gmp(x_new, bt), _gap(x_new, bt)], axis=1))
        if W is None:
            break
        R = _rpad(k)
        h_pad = jnp.zeros((R, NFEAT), jnp.float32).at[:k].set(x_new)
        src3, dst3 = _pad_edges(src, dst, k, R)
        dinv = _degree(k, R, src3, dst3)
        h = _gcn(h_pad, W, b, dinv, src3, dst3)

    out = jax.nn.relu(xs[0]) + jax.nn.relu(xs[1]) + jax.nn.relu(xs[2])
    return (out, augs)


# speedup vs baseline: 3.9079x; 1.0902x over previous
import functools

import jax
import jax.numpy as jnp
import numpy as np
from jax import lax
from jax.experimental import pallas as pl
from jax.experimental.pallas import tpu as pltpu
from jax.experimental.pallas import tpu_sc as plsc

NNODE = 10000
NEDGE = 320000
NFEAT = 128
NGRAPH = 64
RATIO = 0.5

NC, NS, LANES = 2, 16, 16
NW = NC * NS
EC = 128
NCH = -(-NEDGE // (NW * EC))
ZC = 80

_RPAD = {10000: 10240, 5000: 6400, 2500: 3840, 1250: 2560}


def _rpad(k):
    return _RPAD[k]


@functools.lru_cache(maxsize=None)
def _propagate(R, F):
    RT = R // NS
    mesh = plsc.VectorSubcoreMesh(core_axis_name="c", subcore_axis_name="s")

    @functools.partial(
        pl.kernel,
        out_type=jax.ShapeDtypeStruct((NC, R, F), jnp.float32),
        mesh=mesh,
        scratch_types=[
            pltpu.VMEM((2, EC), jnp.int32),
            pltpu.VMEM((2, EC), jnp.int32),
            pltpu.VMEM((2, EC, F), jnp.float32),
            pltpu.VMEM((ZC, F), jnp.float32),
            pltpu.VMEM_SHARED((R, F), jnp.float32),
            pltpu.SemaphoreType.DMA,
            pltpu.SemaphoreType.DMA,
            pltpu.SemaphoreType.DMA,
            pltpu.SemaphoreType.DMA,
            pltpu.SemaphoreType.DMA,
        ],
    )
    def kern(g_h, src_h, dst_h, out_h, sidx, didx, rows, zbuf, acc_sh,
             sem_g, sem_i0, sem_i1, sem_s0, sem_s1):
        cid = lax.axis_index("c")
        sid = lax.axis_index("s")
        tid = sid * NC + cid
        zv = jnp.zeros((LANES,), jnp.float32)

        def zrow(i, _):
            for jf in range(F // LANES):
                zbuf[i, pl.ds(jf * LANES, LANES)] = zv
            return 0

        lax.fori_loop(0, ZC, zrow, 0)

        def zslab(i, _):
            pltpu.sync_copy(zbuf, acc_sh.at[pl.ds(sid * RT + i * ZC, ZC)])
            return 0

        lax.fori_loop(0, RT // ZC, zslab, 0)
        plsc.subcore_barrier()

        sem_i = (sem_i0, sem_i1)
        sem_s = (sem_s0, sem_s1)

        pltpu.sync_copy(src_h.at[tid, 0], sidx.at[0])
        pltpu.sync_copy(dst_h.at[tid, 0], didx.at[0])

        def echunk(j, _):
            for b in range(2):
                @pl.when(j % 2 == b)
                def _():
                    nb = 1 - b
                    @pl.when(j + 1 < NCH)
                    def _():
                        @pl.when(j >= 1)
                        def _():
                            pltpu.make_async_copy(
                                rows.at[nb], acc_sh.at[didx.at[nb]],
                                sem_s[nb]).wait()
                        pltpu.async_copy(src_h.at[tid, j + 1], sidx.at[nb],
                                         sem_i[nb])
                        pltpu.async_copy(dst_h.at[tid, j + 1], didx.at[nb],
                                         sem_i[nb])

                    @pl.when(j >= 1)
                    def _():
                        pltpu.make_async_copy(src_h.at[tid, j], sidx.at[b],
                                              sem_i[b]).wait()
                        pltpu.make_async_copy(dst_h.at[tid, j], didx.at[b],
                                              sem_i[b]).wait()
                    pltpu.async_copy(g_h.at[sidx.at[b]], rows.at[b],
                                     sem_g).wait()
                    pltpu.async_copy(rows.at[b], acc_sh.at[didx.at[b]],
                                     sem_s[b], add=True)
            return 0

        lax.fori_loop(0, NCH, echunk, 0)
        for j in (NCH - 2, NCH - 1):
            b = j % 2
            pltpu.make_async_copy(rows.at[b], acc_sh.at[didx.at[b]],
                                  sem_s[b]).wait()
        plsc.subcore_barrier()

        def wb(i, _):
            sl = pl.ds(sid * RT + i * ZC, ZC)
            pltpu.sync_copy(acc_sh.at[sl], out_h.at[cid, sl])
            return 0

        lax.fori_loop(0, RT // ZC, wb, 0)

    return kern


BR = 256


def _mm_body(x_ref, w_ref, b_ref, ri_ref, ro_ref, o_ref):
    xb = x_ref[...] * ri_ref[...]
    acc = jnp.dot(xb, w_ref[...], preferred_element_type=jnp.float32)
    o_ref[...] = (acc + b_ref[0:1, :]) * ro_ref[...]


def _mm_scale(x, W, b, rs_in, rs_out):
    R, K = x.shape
    N = W.shape[1]
    b8 = jnp.broadcast_to(b[None, :], (8, N))
    return pl.pallas_call(
        _mm_body,
        grid=(R // BR,),
        in_specs=[
            pl.BlockSpec((BR, K), lambda i: (i, 0)),
            pl.BlockSpec((K, N), lambda i: (0, 0)),
            pl.BlockSpec((8, N), lambda i: (0, 0)),
            pl.BlockSpec((BR, 1), lambda i: (i, 0)),
            pl.BlockSpec((BR, 1), lambda i: (i, 0)),
        ],
        out_specs=pl.BlockSpec((BR, N), lambda i: (i, 0)),
        out_shape=jax.ShapeDtypeStruct((R, N), jnp.float32),
    )(x, W, b8, rs_in, rs_out)


def _combine_body(a_ref, g_ref, d_ref, o_ref):
    o_ref[...] = jnp.maximum((a_ref[0] + a_ref[1] + g_ref[...]) * d_ref[...], 0.0)


def _combine(acc, g, dinv):
    R, F = g.shape
    return pl.pallas_call(
        _combine_body,
        grid=(R // BR,),
        in_specs=[
            pl.BlockSpec((NC, BR, F), lambda i: (0, i, 0)),
            pl.BlockSpec((BR, F), lambda i: (i, 0)),
            pl.BlockSpec((BR, 1), lambda i: (i, 0)),
        ],
        out_specs=pl.BlockSpec((BR, F), lambda i: (i, 0)),
        out_shape=jax.ShapeDtypeStruct((R, F), jnp.float32),
    )(acc, g, dinv)


def _pad_edges(src, dst, k, R):
    tot = NW * NCH * EC
    sp = jnp.full((tot,), k, jnp.int32).at[:NEDGE].set(src)
    dp = jnp.full((tot,), k, jnp.int32).at[:NEDGE].set(dst)
    e = jnp.arange(tot, dtype=jnp.int32)
    sp = jnp.where(sp >= k, k + e % (R - k), sp)
    dp = jnp.where(dp >= k, k + (e + 7) % (R - k), dp)
    return sp.reshape(NW, NCH, EC), dp.reshape(NW, NCH, EC)


def _degree(k, R, src3, dst3):
    rows = lax.broadcasted_iota(jnp.int32, (R, 1), 0)
    gdeg = jnp.broadcast_to((rows < k).astype(jnp.float32), (R, NFEAT))
    acc = _propagate(R, NFEAT)(gdeg, src3, dst3)
    deg = acc[0, :, 0] + acc[1, :, 0] + 1.0
    dinv = jnp.where(rows[:, 0] < k, lax.rsqrt(deg), 0.0)
    return dinv[:, None]


def _gcn(x_pad, W, b, dinv, src3, dst3, rs_in=None):
    R = x_pad.shape[0]
    if rs_in is None:
        rs_in = jnp.ones((R, 1), jnp.float32)
    g = _mm_scale(x_pad, W, b, rs_in, dinv)
    acc = _propagate(R, NFEAT)(g, src3, dst3)
    return _combine(acc, g, dinv)


def _gap(x, batch):
    s = jax.ops.segment_sum(x, batch, num_segments=NGRAPH)
    c = jax.ops.segment_sum(jnp.ones((x.shape[0],), jnp.float32), batch,
                            num_segments=NGRAPH)
    return s / jnp.maximum(c, 1.0)[:, None]


def _gmp(x, batch):
    m = jax.ops.segment_max(x, batch, num_segments=NGRAPH)
    c = jax.ops.segment_sum(jnp.ones((x.shape[0],), jnp.float32), batch,
                            num_segments=NGRAPH)
    return jnp.where((c > 0.0)[:, None], m, 0.0)


def _mvpool_jax(y_pad, kprev, src, dst, bt, p, k):
    x = y_pad[:kprev]
    score = jnp.tanh((x @ p) / (jnp.linalg.norm(p) + 1e-12))
    vals, perm = lax.top_k(score, k)
    x_new = x[perm] * vals[:, None]
    newid = jnp.full((kprev + 1,), k, jnp.int32).at[perm].set(
        jnp.arange(k, dtype=jnp.int32))
    return x_new, newid[src], newid[dst], bt[perm]


def kernel(x, edge_index, batch, W1, b1, W2, b2, W3, b3, W7, b7, W8, b8,
           p1, p2, p3):
    src = edge_index[0]
    dst = edge_index[1]
    R0 = _rpad(NNODE)
    x_pad = jnp.zeros((R0, NFEAT), jnp.float32).at[:NNODE].set(x)
    src3, dst3 = _pad_edges(src, dst, NNODE, R0)
    dinv0 = _degree(NNODE, R0, src3, dst3)

    dkey = jax.random.key(7)
    augs = []
    for i in range(10):
        keep = (jax.random.uniform(jax.random.fold_in(dkey, i),
                                   (NNODE,)) > 0.1).astype(jnp.float32)
        keep_pad = jnp.zeros((R0, 1), jnp.float32).at[:NNODE, 0].set(keep)
        y = _gcn(x_pad, W7, b7, dinv0, src3, dst3, rs_in=keep_pad)
        y = _gcn(y, W8, b8, dinv0, src3, dst3)
        yv = y[:NNODE]
        augs.append(_gmp(yv, batch))
        augs.append(_gap(yv, batch))
    augs = jnp.concatenate(augs, axis=1)

    h = _gcn(x_pad, W1, b1, dinv0, src3, dst3)
    ks = [NNODE]
    for r in range(3):
        ks.append(int(np.ceil(RATIO * ks[-1])))
    bt = batch
    xs = []
    for lvl, (p, W, b) in enumerate(
            [(p1, W2, b2), (p2, W3, b3), (p3, None, None)]):
        kprev, k = ks[lvl], ks[lvl + 1]
        x_new, src, dst, bt = _mvpool_jax(h, kprev, src, dst, bt, p, k)
        xs.append(jnp.concatenate([_gmp(x_new, bt), _gap(x_new, bt)], axis=1))
        if W is None:
            break
        R = _rpad(k)
        h_pad = jnp.zeros((R, NFEAT), jnp.float32).at[:k].set(x_new)
        src3, dst3 = _pad_edges(src, dst, k, R)
        dinv = _degree(k, R, src3, dst3)
        h = _gcn(h_pad, W, b, dinv, src3, dst3)

    out = jax.nn.relu(xs[0]) + jax.nn.relu(xs[1]) + jax.nn.relu(xs[2])
    return (out, augs)
